# baseline (speedup 1.0000x reference)
import functools

import jax
import jax.numpy as jnp
from jax import lax
from jax.experimental import pallas as pl
from jax.experimental.pallas import tpu as pltpu
from jax.experimental.pallas import tpu_sc as plsc

N = 128
T = 2048
S = 32
E = N * N
H = 1280
PADW = T + 128
EPT = E // 32

_PREC = lax.Precision.HIGHEST


def _dot(a, b, dims):
    return lax.dot_general(a, b, (dims, ((), ())), precision=_PREC,
                           preferred_element_type=jnp.float32)


def _corr_body(x_ref, xpad_ref, out_ref):
    t = pl.program_id(0)
    xs = pltpu.roll(xpad_ref[...], PADW - 1 - t, 1)[:, :T]
    p = _dot(x_ref[...], xs, ((1,), (1,)))
    out_ref[0] = p * (100.0 / T)


def _corr(x, xpad):
    return pl.pallas_call(
        _corr_body,
        grid=(S,),
        in_specs=[
            pl.BlockSpec((N, T), lambda t: (0, 0)),
            pl.BlockSpec((N, PADW), lambda t: (0, 0)),
        ],
        out_specs=pl.BlockSpec((1, N, N), lambda t: (t, 0, 0)),
        out_shape=jax.ShapeDtypeStruct((S, N, N), jnp.float32),
    )(x, xpad)


def _mlp1_body(tmp_ref, w1_ref, b1_ref, w2_ref, b2_ref, w3_ref, b3_ref,
               w4_ref, b4_ref, out_ref):
    mm = lambda a, b: _dot(a, b, ((1,), (0,)))
    h = jnp.maximum(mm(w1_ref[...], tmp_ref[...]) + b1_ref[...], 0.0)
    h = jnp.maximum(mm(w2_ref[...], h) + b2_ref[...], 0.0)
    h = jnp.maximum(mm(w3_ref[...], h) + b3_ref[...], 0.0)
    out_ref[...] = mm(w4_ref[...], h) + b4_ref[...]


def _mlp1(tmp, w1, b1, w2, b2, w3, b3, w4, b4):
    cols = 2048
    full = lambda a: pl.BlockSpec(a.shape, lambda i: (0,) * a.ndim)
    return pl.pallas_call(
        _mlp1_body,
        grid=(E // cols,),
        in_specs=[
            pl.BlockSpec((S, cols), lambda i: (0, i)),
            full(w1), full(b1), full(w2), full(b2),
            full(w3), full(b3), full(w4), full(b4),
        ],
        out_specs=pl.BlockSpec((1, cols), lambda i: (0, i)),
        out_shape=jax.ShapeDtypeStruct((1, E), jnp.float32),
    )(tmp, w1, b1, w2, b2, w3, b3, w4, b4)


def _sc_route(m_flat, src, dst):
    mesh = plsc.VectorSubcoreMesh(core_axis_name="c", subcore_axis_name="s")

    @functools.partial(
        pl.kernel,
        mesh=mesh,
        compiler_params=pltpu.CompilerParams(needs_layout_passes=False),
        out_type=jax.ShapeDtypeStruct((E,), jnp.float32),
        scratch_types=[
            pltpu.VMEM((E,), jnp.float32),
            pltpu.VMEM((EPT,), jnp.int32),
            pltpu.VMEM((EPT,), jnp.int32),
            pltpu.VMEM((EPT,), jnp.float32),
        ],
    )
    def k(m_hbm, src_hbm, dst_hbm, out_hbm, m_v, src_v, dst_v, acc_v):
        wid = lax.axis_index("s") * 2 + lax.axis_index("c")
        base = wid * EPT
        pltpu.sync_copy(m_hbm, m_v)
        pltpu.sync_copy(src_hbm.at[pl.ds(base, EPT)], src_v)
        pltpu.sync_copy(dst_hbm.at[pl.ds(base, EPT)], dst_v)
        zeros = jnp.zeros((16,), jnp.float32)
        for i in range(EPT // 16):
            acc_v[pl.ds(16 * i, 16)] = zeros
        for i in range(EPT // 16):
            sv = src_v[pl.ds(16 * i, 16)]
            dv = dst_v[pl.ds(16 * i, 16)]
            vals = plsc.load_gather(m_v, [dv * N + sv])
            plsc.addupdate_scatter(acc_v, [dv + (i // 8) * N], vals)
        pltpu.sync_copy(acc_v, out_hbm.at[pl.ds(base, EPT)])

    return k(m_flat, src, dst)


def _mlp2_body(aggT_ref, w5_ref, b5_ref, w6_ref, b6_ref, out_ref):
    uT = jnp.maximum(_dot(w5_ref[...], aggT_ref[...], ((1,), (0,)))
                     + b5_ref[...], 0.0)
    out_ref[...] = _dot(uT, w6_ref[...], ((0,), (1,))) + b6_ref[...]


def _mlp2(aggT, w5, b5, w6, b6):
    full = lambda a: pl.BlockSpec(a.shape, lambda: (0,) * a.ndim)
    return pl.pallas_call(
        _mlp2_body,
        in_specs=[full(aggT), full(w5), full(b5), full(w6), full(b6)],
        out_specs=full(aggT),
        out_shape=jax.ShapeDtypeStruct((N, N), jnp.float32),
    )(aggT, w5, b5, w6, b6)


def _dr1_body(flat_ref, wd1_ref, bd1_ref, out_ref):
    i = pl.program_id(0)
    part = _dot(flat_ref[0], wd1_ref[...], ((1,), (1,)))

    @pl.when(i == 0)
    def _():
        out_ref[...] = jnp.zeros_like(out_ref)

    out_ref[...] += part

    @pl.when(i == pl.num_programs(0) - 1)
    def _():
        out_ref[...] = jnp.maximum(out_ref[...] + bd1_ref[...], 0.0)


def _dr1(flat2d, wd1, bd1):
    kt = 2048
    return pl.pallas_call(
        _dr1_body,
        grid=(E // kt,),
        in_specs=[
            pl.BlockSpec((1, 1, kt), lambda i: (i, 0, 0)),
            pl.BlockSpec((H, kt), lambda i: (0, i)),
            pl.BlockSpec((1, H), lambda i: (0, 0)),
        ],
        out_specs=pl.BlockSpec((1, H), lambda i: (0, 0)),
        out_shape=jax.ShapeDtypeStruct((1, H), jnp.float32),
    )(flat2d, wd1, bd1)


def _dr23_body(h2a_ref, wd2_ref, bd2_ref, wd3_ref, bd3_ref, out_ref, h2f_ref):
    @pl.when(pl.program_id(0) == 0)
    def _():
        h2f_ref[...] = jnp.maximum(
            _dot(h2a_ref[...], wd2_ref[...], ((1,), (1,))) + bd2_ref[...], 0.0)

    out_ref[...] = _dot(h2f_ref[...], wd3_ref[...], ((1,), (1,))) + bd3_ref[...]


def _dr23(h2a, wd2, bd2, wd3, bd3):
    ot = 2048
    return pl.pallas_call(
        _dr23_body,
        grid=(E // ot,),
        in_specs=[
            pl.BlockSpec((1, H), lambda i: (0, 0)),
            pl.BlockSpec((H, H), lambda i: (0, 0)),
            pl.BlockSpec((1, H), lambda i: (0, 0)),
            pl.BlockSpec((ot, H), lambda i: (i, 0)),
            pl.BlockSpec((1, ot), lambda i: (0, i)),
        ],
        out_specs=pl.BlockSpec((1, ot), lambda i: (0, i)),
        out_shape=jax.ShapeDtypeStruct((1, E), jnp.float32),
        scratch_shapes=[pltpu.VMEM((1, H), jnp.float32)],
    )(h2a, wd2, bd2, wd3, bd3)


def kernel(x, edge_index, W1, b1, W2, b2, W3, b3, W4, b4, W5, b5, W6, b6,
           Wd1, bd1, Wd2, bd2, Wd3, bd3):
    src = edge_index[0]
    dst = edge_index[1]
    xpad = jnp.pad(x, ((0, 0), (0, PADW - T)))

    tmp = _corr(x, xpad).reshape(S, E)
    m = _mlp1(tmp, W1, b1.reshape(-1, 1), W2, b2.reshape(-1, 1),
              W3, b3.reshape(-1, 1), W4, b4.reshape(-1, 1))

    aggT = _sc_route(m.reshape(E), src, dst).reshape(N, N)

    high = _mlp2(aggT, W5, b5.reshape(-1, 1), W6, b6.reshape(1, -1))
    h2a = _dr1(high.reshape(8, 1, E // 8), Wd1, bd1.reshape(1, -1))
    out = _dr23(h2a, Wd2, bd2.reshape(1, -1), Wd3, bd3.reshape(1, -1))
    return out.reshape(N, N)


# speedup vs baseline: 1.4008x; 1.4008x over previous
import functools

import jax
import jax.numpy as jnp
from jax import lax
from jax.experimental import pallas as pl
from jax.experimental.pallas import tpu as pltpu
from jax.experimental.pallas import tpu_sc as plsc

N = 128
T = 2048
S = 32
E = N * N
H = 1280
PADW = T + 128
EPT = E // 32

def _dot(a, b, dims, prec=lax.Precision.HIGHEST):
    return lax.dot_general(a, b, (dims, ((), ())), precision=prec,
                           preferred_element_type=jnp.float32)


def _corr_body(x_ref, xpad_ref, out_ref):
    t = pl.program_id(0)
    xs = pltpu.roll(xpad_ref[...], PADW - 1 - t, 1)[:, :T]
    p = _dot(x_ref[...], xs, ((1,), (1,)))
    out_ref[0] = p * (100.0 / T)


def _corr(x, xpad):
    return pl.pallas_call(
        _corr_body,
        grid=(S,),
        in_specs=[
            pl.BlockSpec((N, T), lambda t: (0, 0)),
            pl.BlockSpec((N, PADW), lambda t: (0, 0)),
        ],
        out_specs=pl.BlockSpec((1, N, N), lambda t: (t, 0, 0)),
        out_shape=jax.ShapeDtypeStruct((S, N, N), jnp.float32),
    )(x, xpad)


def _mlp1_body(tmp_ref, w1_ref, b1_ref, w2_ref, b2_ref, w3_ref, b3_ref,
               w4_ref, b4_ref, out_ref):
    mm = lambda a, b: _dot(a, b, ((1,), (0,)), lax.Precision.DEFAULT)
    h = jnp.maximum(mm(w1_ref[...], tmp_ref[...]) + b1_ref[...], 0.0)
    h = jnp.maximum(mm(w2_ref[...], h) + b2_ref[...], 0.0)
    h = jnp.maximum(mm(w3_ref[...], h) + b3_ref[...], 0.0)
    out_ref[...] = mm(w4_ref[...], h) + b4_ref[...]


def _mlp1(tmp, w1, b1, w2, b2, w3, b3, w4, b4):
    cols = 2048
    full = lambda a: pl.BlockSpec(a.shape, lambda i: (0,) * a.ndim)
    return pl.pallas_call(
        _mlp1_body,
        grid=(E // cols,),
        in_specs=[
            pl.BlockSpec((S, cols), lambda i: (0, i)),
            full(w1), full(b1), full(w2), full(b2),
            full(w3), full(b3), full(w4), full(b4),
        ],
        out_specs=pl.BlockSpec((1, cols), lambda i: (0, i)),
        out_shape=jax.ShapeDtypeStruct((1, E), jnp.float32),
    )(tmp, w1, b1, w2, b2, w3, b3, w4, b4)


def _sc_route(m_flat, src, dst):
    mesh = plsc.VectorSubcoreMesh(core_axis_name="c", subcore_axis_name="s")

    @functools.partial(
        pl.kernel,
        mesh=mesh,
        compiler_params=pltpu.CompilerParams(needs_layout_passes=False),
        out_type=jax.ShapeDtypeStruct((E,), jnp.float32),
        scratch_types=[
            pltpu.VMEM((E,), jnp.float32),
            pltpu.VMEM((EPT,), jnp.int32),
            pltpu.VMEM((EPT,), jnp.int32),
            pltpu.VMEM((EPT,), jnp.float32),
        ],
    )
    def k(m_hbm, src_hbm, dst_hbm, out_hbm, m_v, src_v, dst_v, acc_v):
        wid = lax.axis_index("s") * 2 + lax.axis_index("c")
        base = wid * EPT
        pltpu.sync_copy(m_hbm, m_v)
        pltpu.sync_copy(src_hbm.at[pl.ds(base, EPT)], src_v)
        pltpu.sync_copy(dst_hbm.at[pl.ds(base, EPT)], dst_v)
        zeros = jnp.zeros((16,), jnp.float32)
        for i in range(EPT // 16):
            acc_v[pl.ds(16 * i, 16)] = zeros
        for i in range(EPT // 16):
            sv = src_v[pl.ds(16 * i, 16)]
            dv = dst_v[pl.ds(16 * i, 16)]
            vals = plsc.load_gather(m_v, [dv * N + sv])
            plsc.addupdate_scatter(acc_v, [dv + (i // 8) * N], vals)
        pltpu.sync_copy(acc_v, out_hbm.at[pl.ds(base, EPT)])

    return k(m_flat, src, dst)


def _mlp2_body(aggT_ref, w5_ref, b5_ref, w6_ref, b6_ref, out_ref):
    uT = jnp.maximum(_dot(w5_ref[...], aggT_ref[...], ((1,), (0,)))
                     + b5_ref[...], 0.0)
    out_ref[...] = _dot(uT, w6_ref[...], ((0,), (1,))) + b6_ref[...]


def _mlp2(aggT, w5, b5, w6, b6):
    full = lambda a: pl.BlockSpec(a.shape, lambda: (0,) * a.ndim)
    return pl.pallas_call(
        _mlp2_body,
        in_specs=[full(aggT), full(w5), full(b5), full(w6), full(b6)],
        out_specs=full(aggT),
        out_shape=jax.ShapeDtypeStruct((N, N), jnp.float32),
    )(aggT, w5, b5, w6, b6)


def _dr1_body(flat_ref, wd1_ref, bd1_ref, out_ref):
    i = pl.program_id(0)
    part = _dot(flat_ref[0], wd1_ref[...], ((1,), (1,)))

    @pl.when(i == 0)
    def _():
        out_ref[...] = jnp.zeros_like(out_ref)

    out_ref[...] += part

    @pl.when(i == pl.num_programs(0) - 1)
    def _():
        out_ref[...] = jnp.maximum(out_ref[...] + bd1_ref[...], 0.0)


def _dr1(flat2d, wd1, bd1):
    kt = 2048
    return pl.pallas_call(
        _dr1_body,
        grid=(E // kt,),
        in_specs=[
            pl.BlockSpec((1, 1, kt), lambda i: (i, 0, 0)),
            pl.BlockSpec((H, kt), lambda i: (0, i)),
            pl.BlockSpec((1, H), lambda i: (0, 0)),
        ],
        out_specs=pl.BlockSpec((1, H), lambda i: (0, 0)),
        out_shape=jax.ShapeDtypeStruct((1, H), jnp.float32),
    )(flat2d, wd1, bd1)


def _dr23_body(h2a_ref, wd2_ref, bd2_ref, wd3_ref, bd3_ref, out_ref, h2f_ref):
    @pl.when(pl.program_id(0) == 0)
    def _():
        h2f_ref[...] = jnp.maximum(
            _dot(h2a_ref[...], wd2_ref[...], ((1,), (1,))) + bd2_ref[...], 0.0)

    out_ref[...] = _dot(h2f_ref[...], wd3_ref[...], ((1,), (1,))) + bd3_ref[...]


def _dr23(h2a, wd2, bd2, wd3, bd3):
    ot = 2048
    return pl.pallas_call(
        _dr23_body,
        grid=(E // ot,),
        in_specs=[
            pl.BlockSpec((1, H), lambda i: (0, 0)),
            pl.BlockSpec((H, H), lambda i: (0, 0)),
            pl.BlockSpec((1, H), lambda i: (0, 0)),
            pl.BlockSpec((ot, H), lambda i: (i, 0)),
            pl.BlockSpec((1, ot), lambda i: (0, i)),
        ],
        out_specs=pl.BlockSpec((1, ot), lambda i: (0, i)),
        out_shape=jax.ShapeDtypeStruct((1, E), jnp.float32),
        scratch_shapes=[pltpu.VMEM((1, H), jnp.float32)],
    )(h2a, wd2, bd2, wd3, bd3)


def kernel(x, edge_index, W1, b1, W2, b2, W3, b3, W4, b4, W5, b5, W6, b6,
           Wd1, bd1, Wd2, bd2, Wd3, bd3):
    src = edge_index[0]
    dst = edge_index[1]
    xpad = jnp.pad(x, ((0, 0), (0, PADW - T)))

    tmp = _corr(x, xpad).reshape(S, E)
    m = _mlp1(tmp, W1, b1.reshape(-1, 1), W2, b2.reshape(-1, 1),
              W3, b3.reshape(-1, 1), W4, b4.reshape(-1, 1))

    aggT = _sc_route(m.reshape(E), src, dst).reshape(N, N)

    high = _mlp2(aggT, W5, b5.reshape(-1, 1), W6, b6.reshape(1, -1))
    h2a = _dr1(high.reshape(8, 1, E // 8), Wd1, bd1.reshape(1, -1))
    out = _dr23(h2a, Wd2, bd2.reshape(1, -1), Wd3, bd3.reshape(1, -1))
    return out.reshape(N, N)


# speedup vs baseline: 2.9536x; 2.1085x over previous
import functools

import jax
import jax.numpy as jnp
from jax import lax
from jax.experimental import pallas as pl
from jax.experimental.pallas import tpu as pltpu
from jax.experimental.pallas import tpu_sc as plsc

N = 128
T = 2048
S = 32
E = N * N
H = 1280
PADW = T + 128
EPT = E // 32

def _dot(a, b, dims, prec=lax.Precision.DEFAULT):
    return lax.dot_general(a, b, (dims, ((), ())), precision=prec,
                           preferred_element_type=jnp.float32)


def _corr_body(x_ref, xpad_ref, out_ref):
    x = x_ref[...]
    xp = xpad_ref[...]
    for t in range(S):
        xs = pltpu.roll(xp, PADW - 1 - t, 1)[:, :T]
        out_ref[t] = _dot(x, xs, ((1,), (1,))) * (100.0 / T)


def _corr(x, xpad):
    full = lambda a: pl.BlockSpec(a.shape, lambda: (0,) * a.ndim)
    return pl.pallas_call(
        _corr_body,
        in_specs=[full(x), full(xpad)],
        out_specs=pl.BlockSpec((S, N, N), lambda: (0, 0, 0)),
        out_shape=jax.ShapeDtypeStruct((S, N, N), jnp.float32),
    )(x, xpad)


def _mlp1_body(tmp_ref, w1_ref, b1_ref, w2_ref, b2_ref, w3_ref, b3_ref,
               w4_ref, b4_ref, out_ref):
    mm = lambda a, b: _dot(a, b, ((1,), (0,)), lax.Precision.DEFAULT)
    h = jnp.maximum(mm(w1_ref[...], tmp_ref[...]) + b1_ref[...], 0.0)
    h = jnp.maximum(mm(w2_ref[...], h) + b2_ref[...], 0.0)
    h = jnp.maximum(mm(w3_ref[...], h) + b3_ref[...], 0.0)
    out_ref[...] = mm(w4_ref[...], h) + b4_ref[...]


def _mlp1(tmp, w1, b1, w2, b2, w3, b3, w4, b4):
    cols = 2048
    full = lambda a: pl.BlockSpec(a.shape, lambda i: (0,) * a.ndim)
    return pl.pallas_call(
        _mlp1_body,
        grid=(E // cols,),
        in_specs=[
            pl.BlockSpec((S, cols), lambda i: (0, i)),
            full(w1), full(b1), full(w2), full(b2),
            full(w3), full(b3), full(w4), full(b4),
        ],
        out_specs=pl.BlockSpec((1, cols), lambda i: (0, i)),
        out_shape=jax.ShapeDtypeStruct((1, E), jnp.float32),
    )(tmp, w1, b1, w2, b2, w3, b3, w4, b4)


def _sc_route(m_flat, src, dst):
    mesh = plsc.VectorSubcoreMesh(core_axis_name="c", subcore_axis_name="s")

    @functools.partial(
        pl.kernel,
        mesh=mesh,
        compiler_params=pltpu.CompilerParams(needs_layout_passes=False),
        out_type=jax.ShapeDtypeStruct((E,), jnp.float32),
        scratch_types=[
            pltpu.VMEM((E,), jnp.float32),
            pltpu.VMEM((EPT,), jnp.int32),
            pltpu.VMEM((EPT,), jnp.int32),
            pltpu.VMEM((EPT,), jnp.float32),
        ],
    )
    def k(m_hbm, src_hbm, dst_hbm, out_hbm, m_v, src_v, dst_v, acc_v):
        wid = lax.axis_index("s") * 2 + lax.axis_index("c")
        base = wid * EPT
        pltpu.sync_copy(m_hbm, m_v)
        pltpu.sync_copy(src_hbm.at[pl.ds(base, EPT)], src_v)
        pltpu.sync_copy(dst_hbm.at[pl.ds(base, EPT)], dst_v)
        zeros = jnp.zeros((16,), jnp.float32)
        for i in range(EPT // 16):
            acc_v[pl.ds(16 * i, 16)] = zeros
        for i in range(EPT // 16):
            sv = src_v[pl.ds(16 * i, 16)]
            dv = dst_v[pl.ds(16 * i, 16)]
            vals = plsc.load_gather(m_v, [dv * N + sv])
            plsc.addupdate_scatter(acc_v, [dv + (i // 8) * N], vals)
        pltpu.sync_copy(acc_v, out_hbm.at[pl.ds(base, EPT)])

    return k(m_flat, src, dst)


def _mlp2_body(aggT_ref, w5_ref, b5_ref, w6_ref, b6_ref, out_ref):
    uT = jnp.maximum(_dot(w5_ref[...], aggT_ref[...], ((1,), (0,)))
                     + b5_ref[...], 0.0)
    out_ref[...] = _dot(uT, w6_ref[...], ((0,), (1,))) + b6_ref[...]


def _mlp2(aggT, w5, b5, w6, b6):
    full = lambda a: pl.BlockSpec(a.shape, lambda: (0,) * a.ndim)
    return pl.pallas_call(
        _mlp2_body,
        in_specs=[full(aggT), full(w5), full(b5), full(w6), full(b6)],
        out_specs=full(aggT),
        out_shape=jax.ShapeDtypeStruct((N, N), jnp.float32),
    )(aggT, w5, b5, w6, b6)


def _dr_body(flat_ref, wd1_ref, bd1_ref, wd2_ref, bd2_ref, wd3_ref, bd3_ref,
             out_ref, acc_ref, h2f_ref):
    i = pl.program_id(0)

    @pl.when(i == 0)
    def _():
        acc_ref[...] = jnp.zeros_like(acc_ref)

    @pl.when(i < 8)
    def _():
        acc_ref[...] += _dot(flat_ref[0], wd1_ref[...], ((1,), (1,)))

    @pl.when(i == 8)
    def _():
        h2a = jnp.maximum(acc_ref[...] + bd1_ref[...], 0.0)
        h2f_ref[...] = jnp.maximum(
            _dot(h2a, wd2_ref[...], ((1,), (1,))) + bd2_ref[...], 0.0)

    @pl.when(i >= 8)
    def _():
        out_ref[...] = _dot(h2f_ref[...], wd3_ref[...], ((1,), (1,))) + bd3_ref[...]


def _dim_red(flat3d, wd1, bd1, wd2, bd2, wd3, bd3):
    kt = 2048
    mn8 = lambda i: jnp.minimum(i, 7)
    mx8 = lambda i: jnp.maximum(i - 8, 0)
    return pl.pallas_call(
        _dr_body,
        grid=(16,),
        in_specs=[
            pl.BlockSpec((1, 1, kt), lambda i: (mn8(i), 0, 0)),
            pl.BlockSpec((H, kt), lambda i: (0, mn8(i))),
            pl.BlockSpec((1, H), lambda i: (0, 0)),
            pl.BlockSpec((H, H), lambda i: (0, 0)),
            pl.BlockSpec((1, H), lambda i: (0, 0)),
            pl.BlockSpec((kt, H), lambda i: (mx8(i), 0)),
            pl.BlockSpec((1, kt), lambda i: (0, mx8(i))),
        ],
        out_specs=pl.BlockSpec((1, kt), lambda i: (0, mx8(i))),
        out_shape=jax.ShapeDtypeStruct((1, E), jnp.float32),
        scratch_shapes=[pltpu.VMEM((1, H), jnp.float32),
                        pltpu.VMEM((1, H), jnp.float32)],
    )(flat3d, wd1, bd1, wd2, bd2, wd3, bd3)


def kernel(x, edge_index, W1, b1, W2, b2, W3, b3, W4, b4, W5, b5, W6, b6,
           Wd1, bd1, Wd2, bd2, Wd3, bd3):
    src = edge_index[0]
    dst = edge_index[1]
    xpad = jnp.pad(x, ((0, 0), (0, PADW - T)))

    tmp = _corr(x, xpad).reshape(S, E)
    m = _mlp1(tmp, W1, b1.reshape(-1, 1), W2, b2.reshape(-1, 1),
              W3, b3.reshape(-1, 1), W4, b4.reshape(-1, 1))

    aggT = _sc_route(m.reshape(E), src, dst).reshape(N, N)

    high = _mlp2(aggT, W5, b5.reshape(-1, 1), W6, b6.reshape(1, -1))
    out = _dim_red(high.reshape(8, 1, E // 8), Wd1, bd1.reshape(1, -1),
                   Wd2, bd2.reshape(1, -1), Wd3, bd3.reshape(1, -1))
    return out.reshape(N, N)
